# baseline (speedup 1.0000x reference)
import functools

import jax
import jax.numpy as jnp
from jax.experimental import pallas as pl

_BETA = 0.25
_K = 1024
_C = 256
_R = 1024
_N = 32768


def _vq_body(z_ref, emb_ref, embTn_ref, es_ref, zq_ref, inds_ref, loss_ref):
    step = pl.program_id(0)

    zb = z_ref[...]
    dotn = jax.lax.dot_general(
        zb, embTn_ref[...], (((1,), (0,)), ((), ())),
        preferred_element_type=jnp.float32)
    zs_col = jnp.sum(zb * zb, axis=1, keepdims=True)
    d = (zs_col + es_ref[...]) + dotn

    idx = jnp.argmin(d, axis=1, keepdims=True).astype(jnp.int32)
    inds_ref[...] = idx

    iota_k = jax.lax.broadcasted_iota(jnp.int32, (_R, _K), 1)
    onehot = (iota_k == idx).astype(jnp.float32)
    zq = jax.lax.dot_general(
        onehot, emb_ref[...], (((1,), (0,)), ((), ())),
        preferred_element_type=jnp.float32)
    st = zq - zb
    zq_ref[...] = zb + st

    @pl.when(step == 0)
    def _init():
        loss_ref[...] = jnp.zeros_like(loss_ref)

    loss_ref[...] += jnp.sum(st * st, axis=(0, 1), keepdims=True).reshape(1, 1)


@functools.partial(jax.jit, static_argnames=())
def kernel(z, embedding):
    B, C, D, H, W = z.shape
    K = embedding.shape[0]
    xp = jnp.transpose(z, (0, 2, 3, 4, 1)).reshape(-1, C)
    es = jnp.sum(embedding ** 2, axis=1).reshape(1, K)
    embTn = embedding.T * -2.0

    zq2, inds2, loss_acc = pl.pallas_call(
        _vq_body,
        grid=(_N // _R,),
        in_specs=[
            pl.BlockSpec((_R, _C), lambda i: (i, 0)),
            pl.BlockSpec((_K, _C), lambda i: (0, 0)),
            pl.BlockSpec((_C, _K), lambda i: (0, 0)),
            pl.BlockSpec((1, _K), lambda i: (0, 0)),
        ],
        out_specs=[
            pl.BlockSpec((_R, _C), lambda i: (i, 0)),
            pl.BlockSpec((_R, 1), lambda i: (i, 0)),
            pl.BlockSpec((1, 1), lambda i: (0, 0)),
        ],
        out_shape=[
            jax.ShapeDtypeStruct((_N, _C), jnp.float32),
            jax.ShapeDtypeStruct((_N, 1), jnp.int32),
            jax.ShapeDtypeStruct((1, 1), jnp.float32),
        ],
    )(xp, embedding, embTn, es)

    z_q_out = jnp.transpose(zq2.reshape(B, D, H, W, C), (0, 4, 1, 2, 3))
    inds = inds2.reshape(B, D, H, W)
    loss = loss_acc[0, 0] * (_BETA / (B * D * H * W * C))
    return (z_q_out, inds, loss)


# speedup vs baseline: 1.0256x; 1.0256x over previous
import functools

import jax
import jax.numpy as jnp
from jax.experimental import pallas as pl

_BETA = 0.25
_K = 1024
_C = 256
_R = 1024
_N = 32768


def _vq_body(z_ref, emb_ref, embTn_ref, es_ref, zq_ref, inds_ref, loss_ref):
    step = pl.program_id(0)

    zb = z_ref[...]
    dotn = jax.lax.dot_general(
        zb, embTn_ref[...], (((1,), (0,)), ((), ())),
        preferred_element_type=jnp.float32)
    zs_col = jnp.sum(zb * zb, axis=1, keepdims=True)
    d = (zs_col + es_ref[...]) + dotn

    m = jnp.min(d, axis=1, keepdims=True)
    iota_k = jax.lax.broadcasted_iota(jnp.int32, (_R, _K), 1)
    idx = jnp.min(jnp.where(d == m, iota_k, _K), axis=1, keepdims=True)
    inds_ref[...] = idx

    onehot = (iota_k == idx).astype(jnp.float32)
    zq = jax.lax.dot_general(
        onehot, emb_ref[...], (((1,), (0,)), ((), ())),
        preferred_element_type=jnp.float32)
    zq_ref[...] = zb + (zq - zb)

    @pl.when(step == 0)
    def _init():
        loss_ref[...] = jnp.zeros_like(loss_ref)

    loss_ref[...] += jnp.sum(m, axis=(0, 1), keepdims=True).reshape(1, 1)


@functools.partial(jax.jit, static_argnames=())
def kernel(z, embedding):
    B, C, D, H, W = z.shape
    K = embedding.shape[0]
    xp = jnp.transpose(z, (0, 2, 3, 4, 1)).reshape(-1, C)
    es = jnp.sum(embedding ** 2, axis=1).reshape(1, K)
    embTn = embedding.T * -2.0

    zq2, inds2, loss_acc = pl.pallas_call(
        _vq_body,
        grid=(_N // _R,),
        in_specs=[
            pl.BlockSpec((_R, _C), lambda i: (i, 0)),
            pl.BlockSpec((_K, _C), lambda i: (0, 0)),
            pl.BlockSpec((_C, _K), lambda i: (0, 0)),
            pl.BlockSpec((1, _K), lambda i: (0, 0)),
        ],
        out_specs=[
            pl.BlockSpec((_R, _C), lambda i: (i, 0)),
            pl.BlockSpec((_R, 1), lambda i: (i, 0)),
            pl.BlockSpec((1, 1), lambda i: (0, 0)),
        ],
        out_shape=[
            jax.ShapeDtypeStruct((_N, _C), jnp.float32),
            jax.ShapeDtypeStruct((_N, 1), jnp.int32),
            jax.ShapeDtypeStruct((1, 1), jnp.float32),
        ],
    )(xp, embedding, embTn, es)

    z_q_out = jnp.transpose(zq2.reshape(B, D, H, W, C), (0, 4, 1, 2, 3))
    inds = inds2.reshape(B, D, H, W)
    loss = loss_acc[0, 0] * (_BETA / (B * D * H * W * C))
    return (z_q_out, inds, loss)


# speedup vs baseline: 1.1044x; 1.0768x over previous
import functools

import jax
import jax.numpy as jnp
from jax.experimental import pallas as pl

_BETA = 0.25
_K = 1024
_C = 256
_R = 2048
_N = 32768


def _vq_body(z_ref, emb_ref, embTn_ref, es_ref, zq_ref, inds_ref, loss_ref):
    step = pl.program_id(0)

    zb = z_ref[...]
    dotn = jax.lax.dot_general(
        zb, embTn_ref[...], (((1,), (0,)), ((), ())),
        preferred_element_type=jnp.float32)
    zs_col = jnp.sum(zb * zb, axis=1, keepdims=True)
    d = (zs_col + es_ref[...]) + dotn

    m = jnp.min(d, axis=1, keepdims=True)
    iota_k = jax.lax.broadcasted_iota(jnp.int32, (_R, _K), 1)
    idx = jnp.min(jnp.where(d == m, iota_k, _K), axis=1, keepdims=True)
    inds_ref[...] = idx

    onehot = (iota_k == idx).astype(jnp.float32)
    zq = jax.lax.dot_general(
        onehot, emb_ref[...], (((1,), (0,)), ((), ())),
        preferred_element_type=jnp.float32)
    zq_ref[...] = zb + (zq - zb)

    @pl.when(step == 0)
    def _init():
        loss_ref[...] = jnp.zeros_like(loss_ref)

    loss_ref[...] += jnp.sum(m, axis=(0, 1), keepdims=True).reshape(1, 1)


@functools.partial(jax.jit, static_argnames=())
def kernel(z, embedding):
    B, C, D, H, W = z.shape
    K = embedding.shape[0]
    xp = jnp.transpose(z, (0, 2, 3, 4, 1)).reshape(-1, C)
    es = jnp.sum(embedding ** 2, axis=1).reshape(1, K)
    embTn = embedding.T * -2.0

    zq2, inds2, loss_acc = pl.pallas_call(
        _vq_body,
        grid=(_N // _R,),
        in_specs=[
            pl.BlockSpec((_R, _C), lambda i: (i, 0)),
            pl.BlockSpec((_K, _C), lambda i: (0, 0)),
            pl.BlockSpec((_C, _K), lambda i: (0, 0)),
            pl.BlockSpec((1, _K), lambda i: (0, 0)),
        ],
        out_specs=[
            pl.BlockSpec((_R, _C), lambda i: (i, 0)),
            pl.BlockSpec((_R, 1), lambda i: (i, 0)),
            pl.BlockSpec((1, 1), lambda i: (0, 0)),
        ],
        out_shape=[
            jax.ShapeDtypeStruct((_N, _C), jnp.float32),
            jax.ShapeDtypeStruct((_N, 1), jnp.int32),
            jax.ShapeDtypeStruct((1, 1), jnp.float32),
        ],
    )(xp, embedding, embTn, es)

    z_q_out = jnp.transpose(zq2.reshape(B, D, H, W, C), (0, 4, 1, 2, 3))
    inds = inds2.reshape(B, D, H, W)
    loss = loss_acc[0, 0] * (_BETA / (B * D * H * W * C))
    return (z_q_out, inds, loss)


# speedup vs baseline: 1.1397x; 1.0320x over previous
import functools

import jax
import jax.numpy as jnp
from jax.experimental import pallas as pl

_BETA = 0.25
_K = 1024
_C = 256
_R = 4096
_N = 32768


def _vq_body(z_ref, emb_ref, embTn_ref, es_ref, zq_ref, inds_ref, loss_ref):
    step = pl.program_id(0)

    zb = z_ref[...]
    dotn = jax.lax.dot_general(
        zb, embTn_ref[...], (((1,), (0,)), ((), ())),
        preferred_element_type=jnp.float32)
    zs_col = jnp.sum(zb * zb, axis=1, keepdims=True)
    d = (zs_col + es_ref[...]) + dotn

    m = jnp.min(d, axis=1, keepdims=True)
    iota_k = jax.lax.broadcasted_iota(jnp.int32, (_R, _K), 1)
    idx = jnp.min(jnp.where(d == m, iota_k, _K), axis=1, keepdims=True)
    inds_ref[...] = idx

    onehot = (iota_k == idx).astype(jnp.float32)
    zq = jax.lax.dot_general(
        onehot, emb_ref[...], (((1,), (0,)), ((), ())),
        preferred_element_type=jnp.float32)
    zq_ref[...] = zb + (zq - zb)

    @pl.when(step == 0)
    def _init():
        loss_ref[...] = jnp.zeros_like(loss_ref)

    loss_ref[...] += jnp.sum(m, axis=(0, 1), keepdims=True).reshape(1, 1)


@functools.partial(jax.jit, static_argnames=())
def kernel(z, embedding):
    B, C, D, H, W = z.shape
    K = embedding.shape[0]
    xp = jnp.transpose(z, (0, 2, 3, 4, 1)).reshape(-1, C)
    es = jnp.sum(embedding ** 2, axis=1).reshape(1, K)
    embTn = embedding.T * -2.0

    zq2, inds2, loss_acc = pl.pallas_call(
        _vq_body,
        grid=(_N // _R,),
        in_specs=[
            pl.BlockSpec((_R, _C), lambda i: (i, 0)),
            pl.BlockSpec((_K, _C), lambda i: (0, 0)),
            pl.BlockSpec((_C, _K), lambda i: (0, 0)),
            pl.BlockSpec((1, _K), lambda i: (0, 0)),
        ],
        out_specs=[
            pl.BlockSpec((_R, _C), lambda i: (i, 0)),
            pl.BlockSpec((_R, 1), lambda i: (i, 0)),
            pl.BlockSpec((1, 1), lambda i: (0, 0)),
        ],
        out_shape=[
            jax.ShapeDtypeStruct((_N, _C), jnp.float32),
            jax.ShapeDtypeStruct((_N, 1), jnp.int32),
            jax.ShapeDtypeStruct((1, 1), jnp.float32),
        ],
    )(xp, embedding, embTn, es)

    z_q_out = jnp.transpose(zq2.reshape(B, D, H, W, C), (0, 4, 1, 2, 3))
    inds = inds2.reshape(B, D, H, W)
    loss = loss_acc[0, 0] * (_BETA / (B * D * H * W * C))
    return (z_q_out, inds, loss)
